# baseline (speedup 1.0000x reference)
import jax
import jax.numpy as jnp
from jax import lax
from jax.experimental import pallas as pl
from jax.experimental.pallas import tpu as pltpu
from jax.experimental.pallas import tpu_sc as plsc

NC = 2
NS = 16
NW = NC * NS
C = 128


W = 16


def _sc_agg(n_pad, d, k0, k1):
    rpt = n_pad // NS
    nzc = rpt // C
    kmax = max(k0, k1)

    out = jax.ShapeDtypeStruct((NC, n_pad, d), jnp.float32)
    scratch = [
        pltpu.VMEM((kmax, C), jnp.int32),
        pltpu.VMEM((kmax, C), jnp.int32),
        pltpu.VMEM((C, d), jnp.float32),
        pltpu.VMEM_SHARED((n_pad, d), jnp.float32),
    ]

    def body(x_hbm, src_hbm, dst_hbm, agg_out, src_l, dst_l, rows, agg_sh):
        cid = lax.axis_index("c")
        sid = lax.axis_index("s")
        stripe = sid * rpt
        start = jnp.where(cid == 0, sid * k0, NS * k0 + sid * k1)
        kc = jnp.where(cid == 0, k0, k1)

        zero16 = jnp.zeros((16,), jnp.float32)

        def zrow(i, carry):
            for kk in range(d // 16):
                rows[i, pl.ds(kk * 16, 16)] = zero16
            return carry
        lax.fori_loop(0, C, zrow, 0)
        for q in range(nzc):
            pltpu.sync_copy(rows.at[pl.ds(0, C)],
                            agg_sh.at[pl.ds(stripe + q * C, C)])

        pltpu.sync_copy(src_hbm.at[pl.ds(start, kmax)], src_l)
        pltpu.sync_copy(dst_hbm.at[pl.ds(start, kmax)], dst_l)

        plsc.subcore_barrier()

        def chunk(j, carry):
            pltpu.sync_copy(x_hbm.at[src_l.at[j]], rows)
            pltpu.sync_copy(rows, agg_sh.at[dst_l.at[j]], add=True)
            return carry
        lax.fori_loop(0, kc, chunk, 0)

        plsc.subcore_barrier()

        pltpu.sync_copy(agg_sh.at[pl.ds(stripe, rpt)],
                        agg_out.at[cid, pl.ds(stripe, rpt)])

    mesh = plsc.VectorSubcoreMesh(core_axis_name="c", subcore_axis_name="s")
    return pl.kernel(body, out_type=out, mesh=mesh, scratch_types=scratch)


def _sc_deg(n_pad, d, k):
    rpt = n_pad // NS
    nzc = rpt // C

    out = jax.ShapeDtypeStruct((NC, n_pad, d), jnp.float32)
    scratch = [
        pltpu.VMEM((k, C), jnp.int32),
        pltpu.VMEM((C, d), jnp.float32),
        pltpu.VMEM_SHARED((n_pad, d), jnp.float32),
    ]

    def body(dst_hbm, deg_out, dst_l, ones_v, deg_sh):
        cid = lax.axis_index("c")
        sid = lax.axis_index("s")
        wid = sid * NC + cid
        stripe = sid * rpt

        zero16 = jnp.zeros((16,), jnp.float32)
        def zrow(i, carry):
            for kk in range(d // 16):
                ones_v[i, pl.ds(kk * 16, 16)] = zero16
            return carry
        lax.fori_loop(0, C, zrow, 0)
        for q in range(nzc):
            pltpu.sync_copy(ones_v.at[pl.ds(0, C)],
                            deg_sh.at[pl.ds(stripe + q * C, C)])
        one16 = jnp.ones((16,), jnp.float32)
        def orow(i, carry):
            for kk in range(d // 16):
                ones_v[i, pl.ds(kk * 16, 16)] = one16
            return carry
        lax.fori_loop(0, C, orow, 0)

        pltpu.sync_copy(dst_hbm.at[wid], dst_l)
        plsc.subcore_barrier()

        def chunk(j, carry):
            pltpu.sync_copy(ones_v, deg_sh.at[dst_l.at[j]], add=True)
            return carry
        lax.fori_loop(0, k, chunk, 0)

        plsc.subcore_barrier()
        pltpu.sync_copy(deg_sh.at[pl.ds(stripe, rpt)],
                        deg_out.at[cid, pl.ds(stripe, rpt)])

    mesh = plsc.VectorSubcoreMesh(core_axis_name="c", subcore_axis_name="s")
    return pl.kernel(body, out_type=out, mesh=mesh, scratch_types=scratch)


def _dense(n_pad, d, relu):
    rb = 512

    def body(a0, a1, d0, d1, xr, wl, wr, br, o):
        deg = jnp.maximum(d0[...][:, 0:1] + d1[...][:, 0:1], 1.0)
        mean = (a0[...] + a1[...]) / deg
        acc = jnp.dot(mean, wl[...], preferred_element_type=jnp.float32)
        acc = acc + jnp.dot(xr[...], wr[...], preferred_element_type=jnp.float32)
        acc = acc + br[...]
        if relu:
            acc = jnp.maximum(acc, 0.0)
        o[...] = acc

    row_spec = pl.BlockSpec((rb, d), lambda i: (i, 0))
    deg_spec = pl.BlockSpec((rb, d), lambda i: (i, 0))
    full_spec = pl.BlockSpec((d, d), lambda i: (0, 0))
    bias_spec = pl.BlockSpec((1, d), lambda i: (0, 0))
    return pl.pallas_call(
        body,
        grid=(n_pad // rb,),
        in_specs=[row_spec, row_spec, deg_spec, deg_spec, row_spec,
                  full_spec, full_spec, bias_spec],
        out_specs=row_spec,
        out_shape=jax.ShapeDtypeStruct((n_pad, d), jnp.float32),
    )


@jax.jit
def kernel(x, edge_index, W1_l, W1_r, b1, W2_l, W2_r, b2, W3_l, W3_r, b3):
    n, d = x.shape
    e = edge_index.shape[1]

    n_pad = -(-n // (NS * C)) * (NS * C)
    kpair = 16 * (-(-e // (NS * C * 16)))
    t_chunks = NS * kpair
    k0 = int(round(kpair * 0.366 / 8)) * 8
    k1 = kpair - k0
    e_pad = t_chunks * C
    kmax = max(k0, k1)
    src = jnp.concatenate(
        [edge_index[0], jnp.zeros((e_pad - e + kmax * C,), jnp.int32)])
    src = src.reshape(t_chunks + kmax, C)
    dst = jnp.concatenate(
        [edge_index[1], jnp.full((e_pad - e + kmax * C,), n, jnp.int32)])
    dst = dst.reshape(t_chunks + kmax, C)
    xp = jnp.pad(x, ((0, n_pad - n), (0, 0)))

    sc_agg = _sc_agg(n_pad, d, k0, k1)
    dense_relu = _dense(n_pad, d, relu=True)
    dense_lin = _dense(n_pad, d, relu=False)

    kdeg = t_chunks // NW
    deg = _sc_deg(n_pad, d, kdeg)(dst[:t_chunks].reshape(NW, kdeg, C))
    d0, d1 = deg[0], deg[1]

    agg = sc_agg(xp, src, dst)
    h = dense_relu(agg[0], agg[1], d0, d1, xp, W1_l, W1_r, b1.reshape(1, d))

    agg = sc_agg(h, src, dst)
    h = dense_relu(agg[0], agg[1], d0, d1, h, W2_l, W2_r, b2.reshape(1, d))

    agg = sc_agg(h, src, dst)
    out = dense_lin(agg[0], agg[1], d0, d1, h, W3_l, W3_r, b3.reshape(1, d))
    return out[:n]


# speedup vs baseline: 1.1563x; 1.1563x over previous
import jax
import jax.numpy as jnp
from jax import lax
from jax.experimental import pallas as pl
from jax.experimental.pallas import tpu as pltpu
from jax.experimental.pallas import tpu_sc as plsc

NC = 2
NS = 16
NW = NC * NS
C = 128


W = 16


def _sc_agg(n_pad, d, k0, k1):
    rpt = n_pad // NS
    nzc = rpt // C
    kmax = max(k0, k1)

    out = jax.ShapeDtypeStruct((NC, n_pad, d), jnp.float32)
    scratch = [
        pltpu.VMEM((kmax, C), jnp.int32),
        pltpu.VMEM((kmax, C), jnp.int32),
        pltpu.VMEM((C, d), jnp.float32),
        pltpu.VMEM_SHARED((n_pad, d), jnp.float32),
    ]

    def body(x_hbm, src_hbm, dst_hbm, agg_out, src_l, dst_l, rows, agg_sh):
        cid = lax.axis_index("c")
        sid = lax.axis_index("s")
        stripe = sid * rpt
        start = jnp.where(cid == 0, sid * k0, NS * k0 + sid * k1)
        kc = jnp.where(cid == 0, k0, k1)

        zero16 = jnp.zeros((16,), jnp.float32)

        def zrow(i, carry):
            for kk in range(d // 16):
                rows[i, pl.ds(kk * 16, 16)] = zero16
            return carry
        lax.fori_loop(0, C, zrow, 0)
        for q in range(nzc):
            pltpu.sync_copy(rows.at[pl.ds(0, C)],
                            agg_sh.at[pl.ds(stripe + q * C, C)])

        pltpu.sync_copy(src_hbm.at[pl.ds(start, kmax)], src_l)
        pltpu.sync_copy(dst_hbm.at[pl.ds(start, kmax)], dst_l)

        plsc.subcore_barrier()

        def chunk(j, carry):
            pltpu.sync_copy(x_hbm.at[src_l.at[j]], rows)
            pltpu.sync_copy(rows, agg_sh.at[dst_l.at[j]], add=True)
            return carry
        lax.fori_loop(0, kc, chunk, 0)

        plsc.subcore_barrier()

        pltpu.sync_copy(agg_sh.at[pl.ds(stripe, rpt)],
                        agg_out.at[cid, pl.ds(stripe, rpt)])

    mesh = plsc.VectorSubcoreMesh(core_axis_name="c", subcore_axis_name="s")
    return pl.kernel(body, out_type=out, mesh=mesh, scratch_types=scratch)


def _sc_deg(n_pad, d, k):
    rpt = n_pad // NS
    nzc = rpt // C

    out = jax.ShapeDtypeStruct((NC, n_pad, d), jnp.float32)
    scratch = [
        pltpu.VMEM((k, C), jnp.int32),
        pltpu.VMEM((C, d), jnp.float32),
        pltpu.VMEM_SHARED((n_pad, d), jnp.float32),
    ]

    def body(dst_hbm, deg_out, dst_l, ones_v, deg_sh):
        cid = lax.axis_index("c")
        sid = lax.axis_index("s")
        wid = sid * NC + cid
        stripe = sid * rpt

        zero16 = jnp.zeros((16,), jnp.float32)
        def zrow(i, carry):
            for kk in range(d // 16):
                ones_v[i, pl.ds(kk * 16, 16)] = zero16
            return carry
        lax.fori_loop(0, C, zrow, 0)
        for q in range(nzc):
            pltpu.sync_copy(ones_v.at[pl.ds(0, C)],
                            deg_sh.at[pl.ds(stripe + q * C, C)])
        one16 = jnp.ones((16,), jnp.float32)
        def orow(i, carry):
            for kk in range(d // 16):
                ones_v[i, pl.ds(kk * 16, 16)] = one16
            return carry
        lax.fori_loop(0, C, orow, 0)

        pltpu.sync_copy(dst_hbm.at[wid], dst_l)
        plsc.subcore_barrier()

        def chunk(j, carry):
            pltpu.sync_copy(ones_v, deg_sh.at[dst_l.at[j]], add=True)
            return carry
        lax.fori_loop(0, k, chunk, 0)

        plsc.subcore_barrier()
        pltpu.sync_copy(deg_sh.at[pl.ds(stripe, rpt)],
                        deg_out.at[cid, pl.ds(stripe, rpt)])

    mesh = plsc.VectorSubcoreMesh(core_axis_name="c", subcore_axis_name="s")
    return pl.kernel(body, out_type=out, mesh=mesh, scratch_types=scratch)


def _dense(n_pad, d, relu):
    rb = 512

    def body(a0, a1, d0, d1, xr, wl, wr, br, o):
        deg = jnp.maximum(d0[...][:, 0:1] + d1[...][:, 0:1], 1.0)
        mean = (a0[...] + a1[...]) / deg
        acc = jnp.dot(mean, wl[...], preferred_element_type=jnp.float32)
        acc = acc + jnp.dot(xr[...], wr[...], preferred_element_type=jnp.float32)
        acc = acc + br[...]
        if relu:
            acc = jnp.maximum(acc, 0.0)
        o[...] = acc

    row_spec = pl.BlockSpec((rb, d), lambda i: (i, 0))
    deg_spec = pl.BlockSpec((rb, d), lambda i: (i, 0))
    full_spec = pl.BlockSpec((d, d), lambda i: (0, 0))
    bias_spec = pl.BlockSpec((1, d), lambda i: (0, 0))
    return pl.pallas_call(
        body,
        grid=(n_pad // rb,),
        in_specs=[row_spec, row_spec, deg_spec, deg_spec, row_spec,
                  full_spec, full_spec, bias_spec],
        out_specs=row_spec,
        out_shape=jax.ShapeDtypeStruct((n_pad, d), jnp.float32),
    )


@jax.jit
def kernel(x, edge_index, W1_l, W1_r, b1, W2_l, W2_r, b2, W3_l, W3_r, b3):
    n, d = x.shape
    e = edge_index.shape[1]

    n_pad = -(-n // (NS * C)) * (NS * C)
    kpair = 16 * (-(-e // (NS * C * 16)))
    t_chunks = NS * kpair
    k0 = int(round(kpair * 0.634 / 8)) * 8
    k1 = kpair - k0
    e_pad = t_chunks * C
    kmax = max(k0, k1)
    src = jnp.concatenate(
        [edge_index[0], jnp.zeros((e_pad - e + kmax * C,), jnp.int32)])
    src = src.reshape(t_chunks + kmax, C)
    dst = jnp.concatenate(
        [edge_index[1], jnp.full((e_pad - e + kmax * C,), n, jnp.int32)])
    dst = dst.reshape(t_chunks + kmax, C)
    xp = jnp.pad(x, ((0, n_pad - n), (0, 0)))

    sc_agg = _sc_agg(n_pad, d, k0, k1)
    dense_relu = _dense(n_pad, d, relu=True)
    dense_lin = _dense(n_pad, d, relu=False)

    kdeg = t_chunks // NW
    deg = _sc_deg(n_pad, d, kdeg)(dst[:t_chunks].reshape(NW, kdeg, C))
    d0, d1 = deg[0], deg[1]

    agg = sc_agg(xp, src, dst)
    h = dense_relu(agg[0], agg[1], d0, d1, xp, W1_l, W1_r, b1.reshape(1, d))

    agg = sc_agg(h, src, dst)
    h = dense_relu(agg[0], agg[1], d0, d1, h, W2_l, W2_r, b2.reshape(1, d))

    agg = sc_agg(h, src, dst)
    out = dense_lin(agg[0], agg[1], d0, d1, h, W3_l, W3_r, b3.reshape(1, d))
    return out[:n]


# speedup vs baseline: 1.6889x; 1.4607x over previous
import jax
import jax.numpy as jnp
from jax import lax
from jax.experimental import pallas as pl
from jax.experimental.pallas import tpu as pltpu
from jax.experimental.pallas import tpu_sc as plsc

NC = 2
NS = 16
NW = NC * NS
C = 128


W = 16


def _sc_agg(n_pad, d, k):
    rpt = n_pad // NS
    nzc = rpt // C

    out = jax.ShapeDtypeStruct((NC, n_pad, d), jnp.float32)
    scratch = [
        pltpu.VMEM((k, C), jnp.int32),
        pltpu.VMEM((k, C), jnp.int32),
        pltpu.VMEM((C, d), jnp.float32),
        pltpu.VMEM_SHARED((n_pad, d), jnp.float32),
    ]

    def body(x_hbm, src_hbm, dst_hbm, agg_out, src_l, dst_l, rows, agg_sh):
        cid = lax.axis_index("c")
        sid = lax.axis_index("s")
        wid = sid * NC + cid
        stripe = sid * rpt

        zero16 = jnp.zeros((16,), jnp.float32)

        def zrow(i, carry):
            for kk in range(d // 16):
                rows[i, pl.ds(kk * 16, 16)] = zero16
            return carry
        lax.fori_loop(0, C, zrow, 0)
        for q in range(nzc):
            pltpu.sync_copy(rows.at[pl.ds(0, C)],
                            agg_sh.at[pl.ds(stripe + q * C, C)])

        pltpu.sync_copy(src_hbm.at[wid], src_l)
        pltpu.sync_copy(dst_hbm.at[wid], dst_l)

        plsc.subcore_barrier()

        def chunk(j, carry):
            pltpu.sync_copy(x_hbm.at[src_l.at[j]], rows)
            pltpu.sync_copy(rows, agg_sh.at[dst_l.at[j]], add=True)
            return carry
        lax.fori_loop(0, k, chunk, 0)

        plsc.subcore_barrier()

        pltpu.sync_copy(agg_sh.at[pl.ds(stripe, rpt)],
                        agg_out.at[cid, pl.ds(stripe, rpt)])

    mesh = plsc.VectorSubcoreMesh(core_axis_name="c", subcore_axis_name="s")
    return pl.kernel(body, out_type=out, mesh=mesh, scratch_types=scratch)


def _sc_deg(n_pad, d, k):
    rpt = n_pad // NS
    nzc = rpt // C

    out = jax.ShapeDtypeStruct((NC, n_pad, d), jnp.float32)
    scratch = [
        pltpu.VMEM((k, C), jnp.int32),
        pltpu.VMEM((C, d), jnp.float32),
        pltpu.VMEM_SHARED((n_pad, d), jnp.float32),
    ]

    def body(dst_hbm, deg_out, dst_l, ones_v, deg_sh):
        cid = lax.axis_index("c")
        sid = lax.axis_index("s")
        wid = sid * NC + cid
        stripe = sid * rpt

        zero16 = jnp.zeros((16,), jnp.float32)
        def zrow(i, carry):
            for kk in range(d // 16):
                ones_v[i, pl.ds(kk * 16, 16)] = zero16
            return carry
        lax.fori_loop(0, C, zrow, 0)
        for q in range(nzc):
            pltpu.sync_copy(ones_v.at[pl.ds(0, C)],
                            deg_sh.at[pl.ds(stripe + q * C, C)])
        one16 = jnp.ones((16,), jnp.float32)
        def orow(i, carry):
            for kk in range(d // 16):
                ones_v[i, pl.ds(kk * 16, 16)] = one16
            return carry
        lax.fori_loop(0, C, orow, 0)

        pltpu.sync_copy(dst_hbm.at[wid], dst_l)
        plsc.subcore_barrier()

        def chunk(j, carry):
            pltpu.sync_copy(ones_v, deg_sh.at[dst_l.at[j]], add=True)
            return carry
        lax.fori_loop(0, k, chunk, 0)

        plsc.subcore_barrier()
        pltpu.sync_copy(deg_sh.at[pl.ds(stripe, rpt)],
                        deg_out.at[cid, pl.ds(stripe, rpt)])

    mesh = plsc.VectorSubcoreMesh(core_axis_name="c", subcore_axis_name="s")
    return pl.kernel(body, out_type=out, mesh=mesh, scratch_types=scratch)


def _dense(n_pad, d, relu):
    rb = 512

    def body(a0, a1, d0, d1, xr, wl, wr, br, o):
        deg = jnp.maximum(d0[...][:, 0:1] + d1[...][:, 0:1], 1.0)
        mean = (a0[...] + a1[...]) / deg
        acc = jnp.dot(mean, wl[...], preferred_element_type=jnp.float32)
        acc = acc + jnp.dot(xr[...], wr[...], preferred_element_type=jnp.float32)
        acc = acc + br[...]
        if relu:
            acc = jnp.maximum(acc, 0.0)
        o[...] = acc

    row_spec = pl.BlockSpec((rb, d), lambda i: (i, 0))
    deg_spec = pl.BlockSpec((rb, d), lambda i: (i, 0))
    full_spec = pl.BlockSpec((d, d), lambda i: (0, 0))
    bias_spec = pl.BlockSpec((1, d), lambda i: (0, 0))
    return pl.pallas_call(
        body,
        grid=(n_pad // rb,),
        in_specs=[row_spec, row_spec, deg_spec, deg_spec, row_spec,
                  full_spec, full_spec, bias_spec],
        out_specs=row_spec,
        out_shape=jax.ShapeDtypeStruct((n_pad, d), jnp.float32),
    )


@jax.jit
def kernel(x, edge_index, W1_l, W1_r, b1, W2_l, W2_r, b2, W3_l, W3_r, b3):
    n, d = x.shape
    e = edge_index.shape[1]

    n_pad = -(-n // (NS * C)) * (NS * C)
    k = -(-e // (NW * C))
    e_pad = NW * k * C
    src = jnp.concatenate(
        [edge_index[0], jnp.zeros((e_pad - e,), jnp.int32)]).reshape(NW, k, C)
    dst = jnp.concatenate(
        [edge_index[1], jnp.full((e_pad - e,), n, jnp.int32)]).reshape(NW, k, C)
    xp = jnp.pad(x, ((0, n_pad - n), (0, 0)))

    sc_agg = _sc_agg(n_pad, d, k)
    dense_relu = _dense(n_pad, d, relu=True)
    dense_lin = _dense(n_pad, d, relu=False)

    deg = _sc_deg(n_pad, d, k)(dst)
    d0, d1 = deg[0], deg[1]

    agg = sc_agg(xp, src, dst)
    h = dense_relu(agg[0], agg[1], d0, d1, xp, W1_l, W1_r, b1.reshape(1, d))

    agg = sc_agg(h, src, dst)
    h = dense_relu(agg[0], agg[1], d0, d1, h, W2_l, W2_r, b2.reshape(1, d))

    agg = sc_agg(h, src, dst)
    out = dense_lin(agg[0], agg[1], d0, d1, h, W3_l, W3_r, b3.reshape(1, d))
    return out[:n]
